# Initial kernel scaffold; baseline (speedup 1.0000x reference)
import jax
import jax.numpy as jnp
from jax.experimental import pallas as pl


def kernel(adj, embeds, bn_gamma, bn_beta):
    raise NotImplementedError("write your pallas kernel here")


# speedup vs baseline: 1.1767x; 1.1767x over previous
import functools

import jax
import jax.numpy as jnp
from jax.experimental import pallas as pl
from jax.experimental.pallas import tpu as pltpu

_USER = 8192
_ITEM = 8192
_DIM = 64
_TM = 256


def _layer_body(x_ref, gamma_ref, beta_ref, adj_ref,
                ug_ref, ul_ref, ig_ref, il_ref,
                bn_ref, iacc_ref):
    i = pl.program_id(0)
    ni = pl.num_programs(0)

    @pl.when(i == 0)
    def _init():
        x = x_ref[...]
        mean = jnp.mean(x, axis=0, keepdims=True)
        var = jnp.mean((x - mean) ** 2, axis=0, keepdims=True)
        s = gamma_ref[...] / jnp.sqrt(var + 1e-5)
        t = beta_ref[...] - mean * s
        bn_ref[...] = x * s + t
        iacc_ref[...] = jnp.zeros_like(iacc_ref)

    a = adj_ref[...]
    bn_item = bn_ref[_USER:, :]
    bn_user_blk = bn_ref[pl.ds(i * _TM, _TM), :]

    ug = jax.lax.dot_general(
        a, bn_item,
        dimension_numbers=(((1,), (0,)), ((), ())),
        preferred_element_type=jnp.float32)
    ug_ref[...] = ug
    ul_ref[...] = ug + x_ref[pl.ds(i * _TM, _TM), :]

    iacc_ref[...] += jax.lax.dot_general(
        a, bn_user_blk,
        dimension_numbers=(((0,), (0,)), ((), ())),
        preferred_element_type=jnp.float32)

    @pl.when(i == ni - 1)
    def _fin():
        ig = iacc_ref[...]
        ig_ref[...] = ig
        il_ref[...] = ig + x_ref[_USER:, :]


@functools.partial(jax.jit, static_argnames=())
def _fused_layer(adj, x, gamma, beta):
    n_blk = _USER // _TM
    grid = (n_blk,)
    out = pl.pallas_call(
        _layer_body,
        grid=grid,
        in_specs=[
            pl.BlockSpec((_USER + _ITEM, _DIM), lambda i: (0, 0)),
            pl.BlockSpec((1, _DIM), lambda i: (0, 0)),
            pl.BlockSpec((1, _DIM), lambda i: (0, 0)),
            pl.BlockSpec((_TM, _ITEM), lambda i: (i, 0)),
        ],
        out_specs=[
            pl.BlockSpec((_TM, _DIM), lambda i: (i, 0)),
            pl.BlockSpec((_TM, _DIM), lambda i: (i, 0)),
            pl.BlockSpec((_ITEM, _DIM), lambda i: (0, 0)),
            pl.BlockSpec((_ITEM, _DIM), lambda i: (0, 0)),
        ],
        out_shape=[
            jax.ShapeDtypeStruct((_USER, _DIM), jnp.float32),
            jax.ShapeDtypeStruct((_USER, _DIM), jnp.float32),
            jax.ShapeDtypeStruct((_ITEM, _DIM), jnp.float32),
            jax.ShapeDtypeStruct((_ITEM, _DIM), jnp.float32),
        ],
        scratch_shapes=[
            pltpu.VMEM((_USER + _ITEM, _DIM), jnp.float32),
            pltpu.VMEM((_ITEM, _DIM), jnp.float32),
        ],
        compiler_params=pltpu.CompilerParams(
            dimension_semantics=("arbitrary",)),
    )(x, gamma, beta, adj)
    return out


def kernel(adj, embeds, bn_gamma, bn_beta):
    x = embeds
    lats = [embeds]
    gcn_lats = [embeds]
    for layer in range(2):
        ug, ul, ig, il = _fused_layer(
            adj, x,
            bn_gamma[layer][None, :], bn_beta[layer][None, :])
        gcn_lats.append(jnp.concatenate([ug, ig], axis=0))
        x = jnp.concatenate([ul, il], axis=0)
        lats.append(x)
    return (jnp.stack(lats), jnp.stack(gcn_lats))
